# Initial kernel scaffold; baseline (speedup 1.0000x reference)
import jax
import jax.numpy as jnp
from jax.experimental import pallas as pl


def kernel(x, edge_index, W1, b1, W2, b2):
    raise NotImplementedError("write your pallas kernel here")


# speedup vs baseline: 116.0177x; 116.0177x over previous
import functools

import jax
import jax.numpy as jnp
from jax import lax
from jax.experimental import pallas as pl
from jax.experimental.pallas import tpu as pltpu
from jax.experimental.pallas import tpu_sc as plsc

N = 100000
LANES = 128
NROWS = 784
NPAD = NROWS * LANES
NC, NS = 2, 16
NW = NC * NS
E = 3200000
ROWS_W = 784
CH = 56
NOUTER = ROWS_W // CH
EROWS = NW * ROWS_W
EPAD = EROWS * LANES


def _mesh():
    return plsc.VectorSubcoreMesh(core_axis_name="c", subcore_axis_name="s")


def _sc_degree(dst2d, zeros_np, ones_row):
    @functools.partial(
        pl.kernel,
        out_type=jax.ShapeDtypeStruct((NC, NPAD), jnp.float32),
        mesh=_mesh(),
        scratch_types=[
            pltpu.VMEM((CH, LANES), jnp.int32),
            pltpu.VMEM((LANES,), jnp.float32),
            pltpu.VMEM_SHARED((NPAD,), jnp.float32),
        ],
    )
    def deg_kernel(dst_hbm, zeros_hbm, ones_hbm, out_hbm, didx, ones_v, acc_sh):
        c = lax.axis_index("c")
        s = lax.axis_index("s")

        @pl.when(s == 0)
        def _():
            pltpu.sync_copy(zeros_hbm, acc_sh)

        pltpu.sync_copy(ones_hbm, ones_v)
        plsc.subcore_barrier()
        row0 = (c * NS + s) * ROWS_W

        @pl.loop(0, NOUTER)
        def _(t):
            pltpu.sync_copy(dst_hbm.at[pl.ds(row0 + t * CH, CH)], didx)

            @pl.loop(0, CH)
            def _(j):
                pltpu.sync_copy(ones_v, acc_sh.at[didx.at[j]], add=True)

        plsc.subcore_barrier()

        @pl.when(s == 0)
        def _():
            pltpu.sync_copy(acc_sh, out_hbm.at[c])

    return deg_kernel(dst2d, zeros_np, ones_row)


def _sc_scatter(src2d, dst2d, feats, zeros_np):
    F = len(feats)
    scratch = (
        [pltpu.VMEM((CH, LANES), jnp.int32)] * 2
        + [pltpu.VMEM((CH, LANES), jnp.float32)] * F
        + [pltpu.VMEM_SHARED((NPAD,), jnp.float32)] * F
        + [pltpu.VMEM_SHARED((NPAD,), jnp.float32)] * F
    )

    @functools.partial(
        pl.kernel,
        out_type=[jax.ShapeDtypeStruct((NC, NPAD), jnp.float32)] * F,
        mesh=_mesh(),
        scratch_types=scratch,
    )
    def scat_kernel(src_hbm, dst_hbm, *rest):
        feat_hbm = rest[:F]
        zeros_hbm = rest[F]
        outs = rest[F + 1:F + 1 + F]
        sidx, didx = rest[F + 1 + F:F + 3 + F]
        m = rest[F + 3 + F:F + 3 + 2 * F]
        u_sh = rest[F + 3 + 2 * F:F + 3 + 3 * F]
        acc_sh = rest[F + 3 + 3 * F:]
        c = lax.axis_index("c")
        s = lax.axis_index("s")

        @pl.when(s == 0)
        def _():
            for f in range(F):
                pltpu.sync_copy(zeros_hbm, acc_sh[f])
                pltpu.sync_copy(feat_hbm[f], u_sh[f])

        plsc.subcore_barrier()
        row0 = (c * NS + s) * ROWS_W

        @pl.loop(0, NOUTER)
        def _(t):
            pltpu.sync_copy(src_hbm.at[pl.ds(row0 + t * CH, CH)], sidx)
            pltpu.sync_copy(dst_hbm.at[pl.ds(row0 + t * CH, CH)], didx)

            @pl.loop(0, CH)
            def _(j):
                for f in range(F):
                    pltpu.sync_copy(u_sh[f].at[sidx.at[j]], m[f].at[j])
                for f in range(F):
                    pltpu.sync_copy(m[f].at[j], acc_sh[f].at[didx.at[j]], add=True)

        plsc.subcore_barrier()

        @pl.when(s == 0)
        def _():
            for f in range(F):
                pltpu.sync_copy(acc_sh[f], outs[f].at[c])

    return scat_kernel(src2d, dst2d, *feats, zeros_np)


def _round_bf16(a):
    y = lax.bitcast_convert_type(a, jnp.uint32)
    y = (y + jnp.uint32(0x7FFF) + ((y >> 16) & jnp.uint32(1))) \
        & jnp.uint32(0xFFFF0000)
    return lax.bitcast_convert_type(y, jnp.float32)


def _tc_prep(degp, xt):
    def body(degp_ref, xt_ref, dinv_ref, u0_ref, u1_ref):
        deg = degp_ref[0] + degp_ref[1] + 1.0
        dinv = lax.rsqrt(deg)
        dinv = dinv * (1.5 - 0.5 * deg * dinv * dinv)
        dinv = dinv * (1.5 - 0.5 * deg * dinv * dinv)
        dinv_ref[...] = dinv
        u0_ref[...] = xt_ref[0] * dinv
        u1_ref[...] = xt_ref[1] * dinv

    o = jax.ShapeDtypeStruct((NROWS, LANES), jnp.float32)
    return pl.pallas_call(body, out_shape=(o, o, o))(degp, xt)


def _tc_mid(t1p0, t1p1, u0, u1, dinv, W1, b1, W2):
    def _r(a):
        y = lax.bitcast_convert_type(a, jnp.uint32)
        y = (y + jnp.uint32(0x7FFF) + ((y >> 16) & jnp.uint32(1))) \
            & jnp.uint32(0xFFFF0000)
        return lax.bitcast_convert_type(y, jnp.float32)

    def body(t0_ref, t1_ref, u0_ref, u1_ref, dinv_ref, w1_ref, b1_ref,
             w2_ref, v_ref):
        dinv = dinv_ref[...]
        sx0 = _r(dinv * (t0_ref[0] + t0_ref[1] + u0_ref[...]))
        sx1 = _r(dinv * (t1_ref[0] + t1_ref[1] + u1_ref[...]))
        acc = jnp.zeros((NROWS, LANES), jnp.float32)
        for j in range(16):
            hj = jnp.maximum(sx0 * w1_ref[0, j] + sx1 * w1_ref[1, j]
                             + b1_ref[j], 0.0)
            acc = acc + _r(hj) * w2_ref[j, 0]
        v_ref[...] = dinv * acc

    return pl.pallas_call(
        body,
        in_specs=[
            pl.BlockSpec(memory_space=pltpu.VMEM),
            pl.BlockSpec(memory_space=pltpu.VMEM),
            pl.BlockSpec(memory_space=pltpu.VMEM),
            pl.BlockSpec(memory_space=pltpu.VMEM),
            pl.BlockSpec(memory_space=pltpu.VMEM),
            pl.BlockSpec(memory_space=pltpu.SMEM),
            pl.BlockSpec(memory_space=pltpu.SMEM),
            pl.BlockSpec(memory_space=pltpu.SMEM),
        ],
        out_shape=jax.ShapeDtypeStruct((NROWS, LANES), jnp.float32),
    )(t1p0, t1p1, u0, u1, dinv, W1, b1, W2)


def _tc_final(t2p, v, dinv, b2):
    def body(t2_ref, v_ref, dinv_ref, b2_ref, o_ref):
        o_ref[...] = dinv_ref[...] * (t2_ref[0] + t2_ref[1] + v_ref[...]) \
            + b2_ref[0]

    return pl.pallas_call(
        body,
        in_specs=[
            pl.BlockSpec(memory_space=pltpu.VMEM),
            pl.BlockSpec(memory_space=pltpu.VMEM),
            pl.BlockSpec(memory_space=pltpu.VMEM),
            pl.BlockSpec(memory_space=pltpu.SMEM),
        ],
        out_shape=jax.ShapeDtypeStruct((NROWS, LANES), jnp.float32),
    )(t2p, v, dinv, b2)


def kernel(x, edge_index, W1, b1, W2, b2):
    ei = edge_index.astype(jnp.int32)
    pad = EPAD - E
    src2d = jnp.concatenate(
        [ei[0], jnp.zeros((pad,), jnp.int32)]).reshape(EROWS, LANES)
    dst2d = jnp.concatenate(
        [ei[1], jnp.full((pad,), N, jnp.int32)]).reshape(EROWS, LANES)

    xt = jnp.pad(x.T, ((0, 0), (0, NPAD - N))).reshape(2, NROWS, LANES)
    zeros_np = jnp.zeros((NPAD,), jnp.float32)
    ones_row = jnp.ones((LANES,), jnp.float32)

    DBG_SC_DEG = True
    DBG_SC_SCAT = True
    src_p = src2d.reshape(EPAD)
    dst_p = dst2d.reshape(EPAD)
    if DBG_SC_DEG:
        degp = _sc_degree(dst2d, zeros_np, ones_row)
    else:
        deg_j = jnp.zeros((NPAD,), jnp.float32).at[dst_p].add(1.0)
        degp = jnp.stack([deg_j, jnp.zeros((NPAD,), jnp.float32)])
    dinv, u0, u1 = _tc_prep(degp.reshape(NC, NROWS, LANES), xt)

    if DBG_SC_SCAT:
        (t1p0, t1p1) = _sc_scatter(
            src2d, dst2d, (u0.reshape(NPAD), u1.reshape(NPAD)), zeros_np)
    else:
        z2 = jnp.zeros((NPAD,), jnp.float32)
        t1a = z2.at[dst_p].add(u0.reshape(NPAD)[src_p])
        t1b = z2.at[dst_p].add(u1.reshape(NPAD)[src_p])
        t1p0 = jnp.stack([t1a, z2])
        t1p1 = jnp.stack([t1b, z2])
    W1r = _round_bf16(W1)
    W2r = _round_bf16(W2)
    DBG_TC_MID = True
    if DBG_TC_MID:
        v = _tc_mid(t1p0.reshape(NC, NROWS, LANES),
                    t1p1.reshape(NC, NROWS, LANES),
                    u0, u1, dinv, W1r, b1, W2r)
    else:
        dv = dinv.reshape(NPAD)
        sx0 = dv * (t1p0[0] + t1p0[1] + u0.reshape(NPAD))
        sx1 = dv * (t1p1[0] + t1p1[1] + u1.reshape(NPAD))
        sx = jnp.stack([sx0, sx1], axis=1)
        h = jax.nn.relu(sx @ W1 + b1)
        v = ((h @ W2)[:, 0] * dv).reshape(NROWS, LANES)

    if DBG_SC_SCAT:
        (t2p,) = _sc_scatter(src2d, dst2d, (v.reshape(NPAD),), zeros_np)
    else:
        t2a = jnp.zeros((NPAD,), jnp.float32).at[dst_p].add(
            v.reshape(NPAD)[src_p])
        t2p = jnp.stack([t2a, jnp.zeros((NPAD,), jnp.float32)])
    out = _tc_final(t2p.reshape(NC, NROWS, LANES), v, dinv, b2)

    return out.reshape(NPAD)[:N].reshape(N, 1)


# speedup vs baseline: 224.2261x; 1.9327x over previous
import functools

import jax
import jax.numpy as jnp
from jax import lax
from jax.experimental import pallas as pl
from jax.experimental.pallas import tpu as pltpu
from jax.experimental.pallas import tpu_sc as plsc

N = 100000
LANES = 128
NROWS = 784
NPAD = NROWS * LANES
NC, NS = 2, 16
NW = NC * NS
E = 3200000
ROWS_W = 784
CH = 56
NOUTER = ROWS_W // CH
EROWS = NW * ROWS_W
EPAD = EROWS * LANES
CHL = CH * LANES
EW = ROWS_W * LANES


def _mesh():
    return plsc.VectorSubcoreMesh(core_axis_name="c", subcore_axis_name="s")


def _sc_degree(dst1d, zeros_np, ones_b_hbm):
    @functools.partial(
        pl.kernel,
        out_type=jax.ShapeDtypeStruct((NC, NPAD), jnp.float32),
        mesh=_mesh(),
        scratch_types=[
            pltpu.VMEM((CHL,), jnp.int32),
            pltpu.VMEM((CHL,), jnp.float32),
            pltpu.VMEM_SHARED((NPAD,), jnp.float32),
        ],
    )
    def deg_kernel(dst_hbm, zeros_hbm, ones_hbm, out_hbm, didx, ones_b, acc_sh):
        c = lax.axis_index("c")
        s = lax.axis_index("s")

        @pl.when(s == 0)
        def _():
            pltpu.sync_copy(zeros_hbm, acc_sh)

        pltpu.sync_copy(ones_hbm, ones_b)
        plsc.subcore_barrier()
        e0 = (c * NS + s) * EW

        @pl.loop(0, NOUTER)
        def _(t):
            pltpu.sync_copy(dst_hbm.at[pl.ds(e0 + t * CHL, CHL)], didx)
            pltpu.sync_copy(ones_b, acc_sh.at[didx], add=True)

        plsc.subcore_barrier()

        @pl.when(s == 0)
        def _():
            pltpu.sync_copy(acc_sh, out_hbm.at[c])

    return deg_kernel(dst1d, zeros_np, ones_b_hbm)


def _sc_scatter(src1d, dst1d, feats, zeros_np):
    F = len(feats)
    scratch = (
        [pltpu.VMEM((CHL,), jnp.int32)] * 2
        + [pltpu.VMEM((CHL,), jnp.float32)] * F
        + [pltpu.VMEM_SHARED((NPAD,), jnp.float32)] * F
        + [pltpu.VMEM_SHARED((NPAD,), jnp.float32)] * F
    )

    @functools.partial(
        pl.kernel,
        out_type=[jax.ShapeDtypeStruct((NC, NPAD), jnp.float32)] * F,
        mesh=_mesh(),
        scratch_types=scratch,
    )
    def scat_kernel(src_hbm, dst_hbm, *rest):
        feat_hbm = rest[:F]
        zeros_hbm = rest[F]
        outs = rest[F + 1:F + 1 + F]
        sidx, didx = rest[F + 1 + F:F + 3 + F]
        m = rest[F + 3 + F:F + 3 + 2 * F]
        u_sh = rest[F + 3 + 2 * F:F + 3 + 3 * F]
        acc_sh = rest[F + 3 + 3 * F:]
        c = lax.axis_index("c")
        s = lax.axis_index("s")

        @pl.when(s == 0)
        def _():
            for f in range(F):
                pltpu.sync_copy(zeros_hbm, acc_sh[f])
                pltpu.sync_copy(feat_hbm[f], u_sh[f])

        plsc.subcore_barrier()
        e0 = (c * NS + s) * EW

        @pl.loop(0, NOUTER)
        def _(t):
            pltpu.sync_copy(src_hbm.at[pl.ds(e0 + t * CHL, CHL)], sidx)
            pltpu.sync_copy(dst_hbm.at[pl.ds(e0 + t * CHL, CHL)], didx)

            for f in range(F):
                pltpu.sync_copy(u_sh[f].at[sidx], m[f])
            for f in range(F):
                pltpu.sync_copy(m[f], acc_sh[f].at[didx], add=True)

        plsc.subcore_barrier()

        @pl.when(s == 0)
        def _():
            for f in range(F):
                pltpu.sync_copy(acc_sh[f], outs[f].at[c])

    return scat_kernel(src1d, dst1d, *feats, zeros_np)


def _round_bf16(a):
    y = lax.bitcast_convert_type(a, jnp.uint32)
    y = (y + jnp.uint32(0x7FFF) + ((y >> 16) & jnp.uint32(1))) \
        & jnp.uint32(0xFFFF0000)
    return lax.bitcast_convert_type(y, jnp.float32)


def _tc_prep(degp, xt):
    def body(degp_ref, xt_ref, dinv_ref, u0_ref, u1_ref):
        deg = degp_ref[0] + degp_ref[1] + 1.0
        dinv = lax.rsqrt(deg)
        dinv = dinv * (1.5 - 0.5 * deg * dinv * dinv)
        dinv = dinv * (1.5 - 0.5 * deg * dinv * dinv)
        dinv_ref[...] = dinv
        u0_ref[...] = xt_ref[0] * dinv
        u1_ref[...] = xt_ref[1] * dinv

    o = jax.ShapeDtypeStruct((NROWS, LANES), jnp.float32)
    return pl.pallas_call(body, out_shape=(o, o, o))(degp, xt)


def _tc_mid(t1p0, t1p1, u0, u1, dinv, W1, b1, W2):
    def _r(a):
        y = lax.bitcast_convert_type(a, jnp.uint32)
        y = (y + jnp.uint32(0x7FFF) + ((y >> 16) & jnp.uint32(1))) \
            & jnp.uint32(0xFFFF0000)
        return lax.bitcast_convert_type(y, jnp.float32)

    def body(t0_ref, t1_ref, u0_ref, u1_ref, dinv_ref, w1_ref, b1_ref,
             w2_ref, v_ref):
        dinv = dinv_ref[...]
        sx0 = _r(dinv * (t0_ref[0] + t0_ref[1] + u0_ref[...]))
        sx1 = _r(dinv * (t1_ref[0] + t1_ref[1] + u1_ref[...]))
        acc = jnp.zeros((NROWS, LANES), jnp.float32)
        for j in range(16):
            hj = jnp.maximum(sx0 * w1_ref[0, j] + sx1 * w1_ref[1, j]
                             + b1_ref[j], 0.0)
            acc = acc + _r(hj) * w2_ref[j, 0]
        v_ref[...] = dinv * acc

    return pl.pallas_call(
        body,
        in_specs=[
            pl.BlockSpec(memory_space=pltpu.VMEM),
            pl.BlockSpec(memory_space=pltpu.VMEM),
            pl.BlockSpec(memory_space=pltpu.VMEM),
            pl.BlockSpec(memory_space=pltpu.VMEM),
            pl.BlockSpec(memory_space=pltpu.VMEM),
            pl.BlockSpec(memory_space=pltpu.SMEM),
            pl.BlockSpec(memory_space=pltpu.SMEM),
            pl.BlockSpec(memory_space=pltpu.SMEM),
        ],
        out_shape=jax.ShapeDtypeStruct((NROWS, LANES), jnp.float32),
    )(t1p0, t1p1, u0, u1, dinv, W1, b1, W2)


def _tc_final(t2p, v, dinv, b2):
    def body(t2_ref, v_ref, dinv_ref, b2_ref, o_ref):
        o_ref[...] = dinv_ref[...] * (t2_ref[0] + t2_ref[1] + v_ref[...]) \
            + b2_ref[0]

    return pl.pallas_call(
        body,
        in_specs=[
            pl.BlockSpec(memory_space=pltpu.VMEM),
            pl.BlockSpec(memory_space=pltpu.VMEM),
            pl.BlockSpec(memory_space=pltpu.VMEM),
            pl.BlockSpec(memory_space=pltpu.SMEM),
        ],
        out_shape=jax.ShapeDtypeStruct((NROWS, LANES), jnp.float32),
    )(t2p, v, dinv, b2)


def kernel(x, edge_index, W1, b1, W2, b2):
    ei = edge_index.astype(jnp.int32)
    pad = EPAD - E
    src1d = jnp.concatenate([ei[0], jnp.zeros((pad,), jnp.int32)])
    dst1d = jnp.concatenate([ei[1], jnp.full((pad,), N, jnp.int32)])

    xt = jnp.pad(x.T, ((0, 0), (0, NPAD - N))).reshape(2, NROWS, LANES)
    zeros_np = jnp.zeros((NPAD,), jnp.float32)
    ones_row = jnp.ones((CHL,), jnp.float32)

    DBG_SC_DEG = True
    DBG_SC_SCAT = True
    src_p = src1d
    dst_p = dst1d
    if DBG_SC_DEG:
        degp = _sc_degree(dst1d, zeros_np, ones_row)
    else:
        deg_j = jnp.zeros((NPAD,), jnp.float32).at[dst_p].add(1.0)
        degp = jnp.stack([deg_j, jnp.zeros((NPAD,), jnp.float32)])
    dinv, u0, u1 = _tc_prep(degp.reshape(NC, NROWS, LANES), xt)

    if DBG_SC_SCAT:
        (t1p0, t1p1) = _sc_scatter(
            src1d, dst1d, (u0.reshape(NPAD), u1.reshape(NPAD)), zeros_np)
    else:
        z2 = jnp.zeros((NPAD,), jnp.float32)
        t1a = z2.at[dst_p].add(u0.reshape(NPAD)[src_p])
        t1b = z2.at[dst_p].add(u1.reshape(NPAD)[src_p])
        t1p0 = jnp.stack([t1a, z2])
        t1p1 = jnp.stack([t1b, z2])
    W1r = _round_bf16(W1)
    W2r = _round_bf16(W2)
    DBG_TC_MID = True
    if DBG_TC_MID:
        v = _tc_mid(t1p0.reshape(NC, NROWS, LANES),
                    t1p1.reshape(NC, NROWS, LANES),
                    u0, u1, dinv, W1r, b1, W2r)
    else:
        dv = dinv.reshape(NPAD)
        sx0 = dv * (t1p0[0] + t1p0[1] + u0.reshape(NPAD))
        sx1 = dv * (t1p1[0] + t1p1[1] + u1.reshape(NPAD))
        sx = jnp.stack([sx0, sx1], axis=1)
        h = jax.nn.relu(sx @ W1 + b1)
        v = ((h @ W2)[:, 0] * dv).reshape(NROWS, LANES)

    if DBG_SC_SCAT:
        (t2p,) = _sc_scatter(src1d, dst1d, (v.reshape(NPAD),), zeros_np)
    else:
        t2a = jnp.zeros((NPAD,), jnp.float32).at[dst_p].add(
            v.reshape(NPAD)[src_p])
        t2p = jnp.stack([t2a, jnp.zeros((NPAD,), jnp.float32)])
    out = _tc_final(t2p.reshape(NC, NROWS, LANES), v, dinv, b2)

    return out.reshape(NPAD)[:N].reshape(N, 1)


# speedup vs baseline: 235.4834x; 1.0502x over previous
import functools

import jax
import jax.numpy as jnp
from jax import lax
from jax.experimental import pallas as pl
from jax.experimental.pallas import tpu as pltpu
from jax.experimental.pallas import tpu_sc as plsc

N = 100000
LANES = 128
NROWS = 784
NPAD = NROWS * LANES
NC, NS = 2, 16
NW = NC * NS
E = 3200000
ROWS_W = 784
CH = 196
NOUTER = ROWS_W // CH
EROWS = NW * ROWS_W
EPAD = EROWS * LANES
CHL = CH * LANES
EW = ROWS_W * LANES


def _mesh():
    return plsc.VectorSubcoreMesh(core_axis_name="c", subcore_axis_name="s")


def _sc_degree(dst1d, zeros_np, ones_b_hbm):
    @functools.partial(
        pl.kernel,
        out_type=jax.ShapeDtypeStruct((NC, NPAD), jnp.float32),
        mesh=_mesh(),
        scratch_types=[
            pltpu.VMEM((CHL,), jnp.int32),
            pltpu.VMEM((CHL,), jnp.float32),
            pltpu.VMEM_SHARED((NPAD,), jnp.float32),
        ],
    )
    def deg_kernel(dst_hbm, zeros_hbm, ones_hbm, out_hbm, didx, ones_b, acc_sh):
        c = lax.axis_index("c")
        s = lax.axis_index("s")

        @pl.when(s == 0)
        def _():
            pltpu.sync_copy(zeros_hbm, acc_sh)

        pltpu.sync_copy(ones_hbm, ones_b)
        plsc.subcore_barrier()
        e0 = (c * NS + s) * EW

        @pl.loop(0, NOUTER)
        def _(t):
            pltpu.sync_copy(dst_hbm.at[pl.ds(e0 + t * CHL, CHL)], didx)
            pltpu.sync_copy(ones_b, acc_sh.at[didx], add=True)

        plsc.subcore_barrier()

        @pl.when(s == 0)
        def _():
            pltpu.sync_copy(acc_sh, out_hbm.at[c])

    return deg_kernel(dst1d, zeros_np, ones_b_hbm)


def _sc_scatter(src1d, dst1d, feats, zeros_np):
    F = len(feats)
    scratch = (
        [pltpu.VMEM((CHL,), jnp.int32)] * 2
        + [pltpu.VMEM((CHL,), jnp.float32)] * F
        + [pltpu.VMEM_SHARED((NPAD,), jnp.float32)] * F
        + [pltpu.VMEM_SHARED((NPAD,), jnp.float32)] * F
    )

    @functools.partial(
        pl.kernel,
        out_type=[jax.ShapeDtypeStruct((NC, NPAD), jnp.float32)] * F,
        mesh=_mesh(),
        scratch_types=scratch,
    )
    def scat_kernel(src_hbm, dst_hbm, *rest):
        feat_hbm = rest[:F]
        zeros_hbm = rest[F]
        outs = rest[F + 1:F + 1 + F]
        sidx, didx = rest[F + 1 + F:F + 3 + F]
        m = rest[F + 3 + F:F + 3 + 2 * F]
        u_sh = rest[F + 3 + 2 * F:F + 3 + 3 * F]
        acc_sh = rest[F + 3 + 3 * F:]
        c = lax.axis_index("c")
        s = lax.axis_index("s")

        @pl.when(s == 0)
        def _():
            for f in range(F):
                pltpu.sync_copy(zeros_hbm, acc_sh[f])
                pltpu.sync_copy(feat_hbm[f], u_sh[f])

        plsc.subcore_barrier()
        e0 = (c * NS + s) * EW

        @pl.loop(0, NOUTER)
        def _(t):
            pltpu.sync_copy(src_hbm.at[pl.ds(e0 + t * CHL, CHL)], sidx)
            pltpu.sync_copy(dst_hbm.at[pl.ds(e0 + t * CHL, CHL)], didx)

            for f in range(F):
                pltpu.sync_copy(u_sh[f].at[sidx], m[f])
            for f in range(F):
                pltpu.sync_copy(m[f], acc_sh[f].at[didx], add=True)

        plsc.subcore_barrier()

        @pl.when(s == 0)
        def _():
            for f in range(F):
                pltpu.sync_copy(acc_sh[f], outs[f].at[c])

    return scat_kernel(src1d, dst1d, *feats, zeros_np)


def _round_bf16(a):
    y = lax.bitcast_convert_type(a, jnp.uint32)
    y = (y + jnp.uint32(0x7FFF) + ((y >> 16) & jnp.uint32(1))) \
        & jnp.uint32(0xFFFF0000)
    return lax.bitcast_convert_type(y, jnp.float32)


def _tc_prep(degp, xt):
    def body(degp_ref, xt_ref, dinv_ref, u0_ref, u1_ref):
        deg = degp_ref[0] + degp_ref[1] + 1.0
        dinv = lax.rsqrt(deg)
        dinv = dinv * (1.5 - 0.5 * deg * dinv * dinv)
        dinv = dinv * (1.5 - 0.5 * deg * dinv * dinv)
        dinv_ref[...] = dinv
        u0_ref[...] = xt_ref[0] * dinv
        u1_ref[...] = xt_ref[1] * dinv

    o = jax.ShapeDtypeStruct((NROWS, LANES), jnp.float32)
    return pl.pallas_call(body, out_shape=(o, o, o))(degp, xt)


def _tc_mid(t1p0, t1p1, u0, u1, dinv, W1, b1, W2):
    def _r(a):
        y = lax.bitcast_convert_type(a, jnp.uint32)
        y = (y + jnp.uint32(0x7FFF) + ((y >> 16) & jnp.uint32(1))) \
            & jnp.uint32(0xFFFF0000)
        return lax.bitcast_convert_type(y, jnp.float32)

    def body(t0_ref, t1_ref, u0_ref, u1_ref, dinv_ref, w1_ref, b1_ref,
             w2_ref, v_ref):
        dinv = dinv_ref[...]
        sx0 = _r(dinv * (t0_ref[0] + t0_ref[1] + u0_ref[...]))
        sx1 = _r(dinv * (t1_ref[0] + t1_ref[1] + u1_ref[...]))
        acc = jnp.zeros((NROWS, LANES), jnp.float32)
        for j in range(16):
            hj = jnp.maximum(sx0 * w1_ref[0, j] + sx1 * w1_ref[1, j]
                             + b1_ref[j], 0.0)
            acc = acc + _r(hj) * w2_ref[j, 0]
        v_ref[...] = dinv * acc

    return pl.pallas_call(
        body,
        in_specs=[
            pl.BlockSpec(memory_space=pltpu.VMEM),
            pl.BlockSpec(memory_space=pltpu.VMEM),
            pl.BlockSpec(memory_space=pltpu.VMEM),
            pl.BlockSpec(memory_space=pltpu.VMEM),
            pl.BlockSpec(memory_space=pltpu.VMEM),
            pl.BlockSpec(memory_space=pltpu.SMEM),
            pl.BlockSpec(memory_space=pltpu.SMEM),
            pl.BlockSpec(memory_space=pltpu.SMEM),
        ],
        out_shape=jax.ShapeDtypeStruct((NROWS, LANES), jnp.float32),
    )(t1p0, t1p1, u0, u1, dinv, W1, b1, W2)


def _tc_final(t2p, v, dinv, b2):
    def body(t2_ref, v_ref, dinv_ref, b2_ref, o_ref):
        o_ref[...] = dinv_ref[...] * (t2_ref[0] + t2_ref[1] + v_ref[...]) \
            + b2_ref[0]

    return pl.pallas_call(
        body,
        in_specs=[
            pl.BlockSpec(memory_space=pltpu.VMEM),
            pl.BlockSpec(memory_space=pltpu.VMEM),
            pl.BlockSpec(memory_space=pltpu.VMEM),
            pl.BlockSpec(memory_space=pltpu.SMEM),
        ],
        out_shape=jax.ShapeDtypeStruct((NROWS, LANES), jnp.float32),
    )(t2p, v, dinv, b2)


def kernel(x, edge_index, W1, b1, W2, b2):
    ei = edge_index.astype(jnp.int32)
    pad = EPAD - E
    src1d = jnp.concatenate([ei[0], jnp.zeros((pad,), jnp.int32)])
    dst1d = jnp.concatenate([ei[1], jnp.full((pad,), N, jnp.int32)])

    xt = jnp.pad(x.T, ((0, 0), (0, NPAD - N))).reshape(2, NROWS, LANES)
    zeros_np = jnp.zeros((NPAD,), jnp.float32)
    ones_row = jnp.ones((CHL,), jnp.float32)

    DBG_SC_DEG = True
    DBG_SC_SCAT = True
    src_p = src1d
    dst_p = dst1d
    if DBG_SC_DEG:
        degp = _sc_degree(dst1d, zeros_np, ones_row)
    else:
        deg_j = jnp.zeros((NPAD,), jnp.float32).at[dst_p].add(1.0)
        degp = jnp.stack([deg_j, jnp.zeros((NPAD,), jnp.float32)])
    dinv, u0, u1 = _tc_prep(degp.reshape(NC, NROWS, LANES), xt)

    if DBG_SC_SCAT:
        (t1p0, t1p1) = _sc_scatter(
            src1d, dst1d, (u0.reshape(NPAD), u1.reshape(NPAD)), zeros_np)
    else:
        z2 = jnp.zeros((NPAD,), jnp.float32)
        t1a = z2.at[dst_p].add(u0.reshape(NPAD)[src_p])
        t1b = z2.at[dst_p].add(u1.reshape(NPAD)[src_p])
        t1p0 = jnp.stack([t1a, z2])
        t1p1 = jnp.stack([t1b, z2])
    W1r = _round_bf16(W1)
    W2r = _round_bf16(W2)
    DBG_TC_MID = True
    if DBG_TC_MID:
        v = _tc_mid(t1p0.reshape(NC, NROWS, LANES),
                    t1p1.reshape(NC, NROWS, LANES),
                    u0, u1, dinv, W1r, b1, W2r)
    else:
        dv = dinv.reshape(NPAD)
        sx0 = dv * (t1p0[0] + t1p0[1] + u0.reshape(NPAD))
        sx1 = dv * (t1p1[0] + t1p1[1] + u1.reshape(NPAD))
        sx = jnp.stack([sx0, sx1], axis=1)
        h = jax.nn.relu(sx @ W1 + b1)
        v = ((h @ W2)[:, 0] * dv).reshape(NROWS, LANES)

    if DBG_SC_SCAT:
        (t2p,) = _sc_scatter(src1d, dst1d, (v.reshape(NPAD),), zeros_np)
    else:
        t2a = jnp.zeros((NPAD,), jnp.float32).at[dst_p].add(
            v.reshape(NPAD)[src_p])
        t2p = jnp.stack([t2a, jnp.zeros((NPAD,), jnp.float32)])
    out = _tc_final(t2p.reshape(NC, NROWS, LANES), v, dinv, b2)

    return out.reshape(NPAD)[:N].reshape(N, 1)
